# baseline (speedup 1.0000x reference)
from functools import partial

import jax
import jax.numpy as jnp
from jax.experimental import pallas as pl
from jax.experimental.pallas import tpu as pltpu

_F32 = jnp.float32


def _silu(v):
    return v * jax.nn.sigmoid(v)


def _front_kernel(x_ref, Wu_ref, Wx_ref, Wdt_ref, cw_ref, cb_ref, bdt_ref,
                  u_ref, dt_ref, BT_ref, CT_ref, carry_ref,
                  *, dt_rank, n_state, d_conv):
    j = pl.program_id(1)

    @pl.when(j == 0)
    def _():
        carry_ref[...] = jnp.zeros_like(carry_ref)

    x = x_ref[0]
    u_raw = jnp.dot(x, Wu_ref[...], preferred_element_type=_F32)

    Lt = u_raw.shape[0]
    full = jnp.concatenate([carry_ref[...], u_raw], axis=0)
    carry_ref[...] = full[Lt:Lt + 8]
    acc = jnp.zeros_like(u_raw) + cb_ref[...]
    for k in range(d_conv):
        off = 8 - (d_conv - 1) + k
        acc = acc + full[off:off + Lt] * cw_ref[k:k + 1, :]
    u = _silu(acc)
    u_ref[...] = u

    xdbl = jnp.dot(u.astype(jnp.bfloat16), Wx_ref[...],
                   preferred_element_type=_F32)
    dt_low = xdbl[:, :dt_rank]
    dtv = jax.nn.softplus(
        jnp.dot(dt_low.astype(jnp.bfloat16), Wdt_ref[...],
                preferred_element_type=_F32) + bdt_ref[...])
    dt_ref[...] = dtv
    BT_ref[...] = xdbl[:, dt_rank:dt_rank + n_state].T
    CT_ref[...] = xdbl[:, dt_rank + n_state:dt_rank + 2 * n_state].T


def _scan_kernel(u_ref, dt_ref, BT_ref, CT_ref, A_ref, D_ref,
                 y_ref, h_ref):
    j = pl.program_id(2)

    @pl.when(j == 0)
    def _():
        h_ref[...] = jnp.zeros_like(h_ref)

    u = u_ref[...]
    dt = dt_ref[...]
    A = A_ref[...]
    dtu = dt * u
    BT = BT_ref[...]
    CT = CT_ref[...]
    h = h_ref[...]
    Tc = u.shape[0]
    for t in range(Tc):
        dA = jnp.exp2(dt[t:t + 1, :] * A)
        h = dA * h + dtu[t:t + 1, :] * BT[:, t:t + 1]
        y_ref[t:t + 1, :] = jnp.sum(h * CT[:, t:t + 1], axis=0, keepdims=True)
    h_ref[...] = h
    y_ref[...] = y_ref[...] + u * D_ref[...]


def _out_kernel(x_ref, Wr_ref, y_ref, Wo_ref, o_ref):
    res = jnp.dot(x_ref[0], Wr_ref[...], preferred_element_type=_F32)
    g = y_ref[...] * _silu(res)
    o_ref[0] = jnp.dot(g.astype(jnp.bfloat16), Wo_ref[...],
                       preferred_element_type=_F32)


def kernel(x, W_in, conv_w, conv_b, W_xproj, W_dt, b_dt, A_log, D, W_out):
    B, L, d_model = x.shape
    d_inner, d_conv = conv_w.shape
    dt_rank = W_dt.shape[0]
    n_state = A_log.shape[1]

    Lt = min(512, L)
    J1 = L // Lt
    Tc = min(256, L)
    J2 = L // Tc
    dblk = min(512, d_inner)
    Gd = d_inner // dblk

    bf16 = jnp.bfloat16
    xh = x.astype(bf16)
    Wu = W_in[:, :d_inner].astype(bf16)
    Wr = W_in[:, d_inner:].astype(bf16)
    Wxh = W_xproj.astype(bf16)
    Wdth = W_dt.astype(bf16)
    Woh = W_out.astype(bf16)
    cwT = conv_w.T
    cb = conv_b[None, :]
    bdt = b_dt[None, :]
    LOG2E = 1.4426950408889634
    AT = (-jnp.exp(A_log) * LOG2E).T
    Dc = jnp.concatenate([D] * B)[None, :]

    cp = pltpu.CompilerParams(
        dimension_semantics=("parallel", "arbitrary"),
        vmem_limit_bytes=56 * 1024 * 1024,
    )

    u_c, dt_c, BT, CT = pl.pallas_call(
        partial(_front_kernel, dt_rank=dt_rank, n_state=n_state, d_conv=d_conv),
        grid=(B, J1),
        in_specs=[
            pl.BlockSpec((1, Lt, d_model), lambda b, j: (b, j, 0)),
            pl.BlockSpec((d_model, d_inner), lambda b, j: (0, 0)),
            pl.BlockSpec((d_inner, dt_rank + 2 * n_state), lambda b, j: (0, 0)),
            pl.BlockSpec((dt_rank, d_inner), lambda b, j: (0, 0)),
            pl.BlockSpec((d_conv, d_inner), lambda b, j: (0, 0)),
            pl.BlockSpec((1, d_inner), lambda b, j: (0, 0)),
            pl.BlockSpec((1, d_inner), lambda b, j: (0, 0)),
        ],
        out_specs=[
            pl.BlockSpec((Lt, d_inner), lambda b, j: (j, b)),
            pl.BlockSpec((Lt, d_inner), lambda b, j: (j, b)),
            pl.BlockSpec((n_state, Lt), lambda b, j: (b, j)),
            pl.BlockSpec((n_state, Lt), lambda b, j: (b, j)),
        ],
        out_shape=[
            jax.ShapeDtypeStruct((L, B * d_inner), _F32),
            jax.ShapeDtypeStruct((L, B * d_inner), _F32),
            jax.ShapeDtypeStruct((B * n_state, L), _F32),
            jax.ShapeDtypeStruct((B * n_state, L), _F32),
        ],
        scratch_shapes=[pltpu.VMEM((8, d_inner), _F32)],
        compiler_params=cp,
        name="mamba_front",
    )(xh, Wu, Wxh, Wdth, cwT, cb, bdt)

    Gh = (B * Gd) // 2
    cp_scan = pltpu.CompilerParams(
        dimension_semantics=("parallel", "arbitrary", "arbitrary"),
        vmem_limit_bytes=56 * 1024 * 1024,
    )
    y2 = pl.pallas_call(
        _scan_kernel,
        grid=(2, Gh, J2),
        in_specs=[
            pl.BlockSpec((Tc, dblk), lambda c, g, j: (j, c * Gh + g)),
            pl.BlockSpec((Tc, dblk), lambda c, g, j: (j, c * Gh + g)),
            pl.BlockSpec((n_state, Tc), lambda c, g, j: ((c * Gh + g) // Gd, j)),
            pl.BlockSpec((n_state, Tc), lambda c, g, j: ((c * Gh + g) // Gd, j)),
            pl.BlockSpec((n_state, dblk), lambda c, g, j: (0, (c * Gh + g) % Gd)),
            pl.BlockSpec((1, dblk), lambda c, g, j: (0, c * Gh + g)),
        ],
        out_specs=pl.BlockSpec((Tc, dblk), lambda c, g, j: (j, c * Gh + g)),
        out_shape=jax.ShapeDtypeStruct((L, B * d_inner), _F32),
        scratch_shapes=[pltpu.VMEM((n_state, dblk), _F32)],
        compiler_params=cp_scan,
        name="mamba_scan",
    )(u_c, dt_c, BT, CT, AT, Dc)

    o = pl.pallas_call(
        _out_kernel,
        grid=(B, J1),
        in_specs=[
            pl.BlockSpec((1, Lt, d_model), lambda b, j: (b, j, 0)),
            pl.BlockSpec((d_model, d_inner), lambda b, j: (0, 0)),
            pl.BlockSpec((Lt, d_inner), lambda b, j: (j, b)),
            pl.BlockSpec((d_inner, d_model), lambda b, j: (0, 0)),
        ],
        out_specs=pl.BlockSpec((1, Lt, d_model), lambda b, j: (b, j, 0)),
        out_shape=jax.ShapeDtypeStruct((B, L, d_model), _F32),
        compiler_params=cp,
        name="mamba_out",
    )(xh, Wr, y2, Woh)

    return o


# speedup vs baseline: 1.0704x; 1.0704x over previous
from functools import partial

import jax
import jax.numpy as jnp
from jax.experimental import pallas as pl
from jax.experimental.pallas import tpu as pltpu

_F32 = jnp.float32


def _silu(v):
    return v * jax.nn.sigmoid(v)


def _front_kernel(x_ref, Wu_ref, Wx_ref, Wdt_ref, cw_ref, cb_ref, bdt_ref,
                  u_ref, dt_ref, BT_ref, CT_ref, carry_ref,
                  *, dt_rank, n_state, d_conv):
    j = pl.program_id(1)

    @pl.when(j == 0)
    def _():
        carry_ref[...] = jnp.zeros_like(carry_ref)

    x = x_ref[0]
    u_raw = jnp.dot(x, Wu_ref[...], preferred_element_type=_F32)

    Lt = u_raw.shape[0]
    full = jnp.concatenate([carry_ref[...], u_raw], axis=0)
    carry_ref[...] = full[Lt:Lt + 8]
    acc = jnp.zeros_like(u_raw) + cb_ref[...]
    for k in range(d_conv):
        off = 8 - (d_conv - 1) + k
        acc = acc + full[off:off + Lt] * cw_ref[k:k + 1, :]
    u = _silu(acc)
    u_ref[...] = u

    xdbl = jnp.dot(u, Wx_ref[...], preferred_element_type=_F32)
    dt_low = xdbl[:, :dt_rank]
    dtv = jax.nn.softplus(
        jnp.dot(dt_low, Wdt_ref[...], preferred_element_type=_F32) + bdt_ref[...])
    dt_ref[...] = dtv
    BT_ref[...] = xdbl[:, dt_rank:dt_rank + n_state].T
    CT_ref[...] = xdbl[:, dt_rank + n_state:dt_rank + 2 * n_state].T


def _scan_kernel(u_ref, dt_ref, BT_ref, CT_ref, A_ref, D_ref,
                 y_ref, h_ref):
    j = pl.program_id(2)

    @pl.when(j == 0)
    def _():
        h_ref[...] = jnp.zeros_like(h_ref)

    u = u_ref[...]
    dt = dt_ref[...]
    A = A_ref[...]
    dtu = dt * u
    BT = BT_ref[...]
    CT = CT_ref[...]
    h = h_ref[...]
    Tc = u.shape[0]
    for t in range(Tc):
        dA = jnp.exp2(dt[t:t + 1, :] * A)
        h = dA * h + dtu[t:t + 1, :] * BT[:, t:t + 1]
        y_ref[t:t + 1, :] = jnp.sum(h * CT[:, t:t + 1], axis=0, keepdims=True)
    h_ref[...] = h
    y_ref[...] = y_ref[...] + u * D_ref[...]


def _out_kernel(x_ref, Wr_ref, y_ref, Wo_ref, o_ref):
    res = jnp.dot(x_ref[0], Wr_ref[...], preferred_element_type=_F32)
    g = y_ref[...] * _silu(res)
    o_ref[0] = jnp.dot(g, Wo_ref[...], preferred_element_type=_F32)


def kernel(x, W_in, conv_w, conv_b, W_xproj, W_dt, b_dt, A_log, D, W_out):
    B, L, d_model = x.shape
    d_inner, d_conv = conv_w.shape
    dt_rank = W_dt.shape[0]
    n_state = A_log.shape[1]

    Lt = min(512, L)
    J1 = L // Lt
    Tc = min(256, L)
    J2 = L // Tc
    dblk = min(512, d_inner)
    Gd = d_inner // dblk

    cwT = conv_w.T
    cb = conv_b[None, :]
    bdt = b_dt[None, :]
    LOG2E = 1.4426950408889634
    AT = (-jnp.exp(A_log) * LOG2E).T
    Dc = jnp.concatenate([D] * B)[None, :]

    cp = pltpu.CompilerParams(
        dimension_semantics=("parallel", "arbitrary"),
        vmem_limit_bytes=56 * 1024 * 1024,
    )

    u_c, dt_c, BT, CT = pl.pallas_call(
        partial(_front_kernel, dt_rank=dt_rank, n_state=n_state, d_conv=d_conv),
        grid=(B, J1),
        in_specs=[
            pl.BlockSpec((1, Lt, d_model), lambda b, j: (b, j, 0)),
            pl.BlockSpec((d_model, d_inner), lambda b, j: (0, 0)),
            pl.BlockSpec((d_inner, dt_rank + 2 * n_state), lambda b, j: (0, 0)),
            pl.BlockSpec((dt_rank, d_inner), lambda b, j: (0, 0)),
            pl.BlockSpec((d_conv, d_inner), lambda b, j: (0, 0)),
            pl.BlockSpec((1, d_inner), lambda b, j: (0, 0)),
            pl.BlockSpec((1, d_inner), lambda b, j: (0, 0)),
        ],
        out_specs=[
            pl.BlockSpec((Lt, d_inner), lambda b, j: (j, b)),
            pl.BlockSpec((Lt, d_inner), lambda b, j: (j, b)),
            pl.BlockSpec((n_state, Lt), lambda b, j: (b, j)),
            pl.BlockSpec((n_state, Lt), lambda b, j: (b, j)),
        ],
        out_shape=[
            jax.ShapeDtypeStruct((L, B * d_inner), _F32),
            jax.ShapeDtypeStruct((L, B * d_inner), _F32),
            jax.ShapeDtypeStruct((B * n_state, L), _F32),
            jax.ShapeDtypeStruct((B * n_state, L), _F32),
        ],
        scratch_shapes=[pltpu.VMEM((8, d_inner), _F32)],
        compiler_params=cp,
        name="mamba_front",
    )(x, W_in, W_xproj, W_dt, cwT, cb, bdt)

    Gh = (B * Gd) // 2
    cp_scan = pltpu.CompilerParams(
        dimension_semantics=("parallel", "arbitrary", "arbitrary"),
        vmem_limit_bytes=56 * 1024 * 1024,
    )
    y2 = pl.pallas_call(
        _scan_kernel,
        grid=(2, Gh, J2),
        in_specs=[
            pl.BlockSpec((Tc, dblk), lambda c, g, j: (j, c * Gh + g)),
            pl.BlockSpec((Tc, dblk), lambda c, g, j: (j, c * Gh + g)),
            pl.BlockSpec((n_state, Tc), lambda c, g, j: ((c * Gh + g) // Gd, j)),
            pl.BlockSpec((n_state, Tc), lambda c, g, j: ((c * Gh + g) // Gd, j)),
            pl.BlockSpec((n_state, dblk), lambda c, g, j: (0, (c * Gh + g) % Gd)),
            pl.BlockSpec((1, dblk), lambda c, g, j: (0, c * Gh + g)),
        ],
        out_specs=pl.BlockSpec((Tc, dblk), lambda c, g, j: (j, c * Gh + g)),
        out_shape=jax.ShapeDtypeStruct((L, B * d_inner), _F32),
        scratch_shapes=[pltpu.VMEM((n_state, dblk), _F32)],
        compiler_params=cp_scan,
        name="mamba_scan",
    )(u_c, dt_c, BT, CT, AT, Dc)

    o = pl.pallas_call(
        _out_kernel,
        grid=(B, J1),
        in_specs=[
            pl.BlockSpec((1, Lt, d_model), lambda b, j: (b, j, 0)),
            pl.BlockSpec((d_model, d_inner), lambda b, j: (0, 1)),
            pl.BlockSpec((Lt, d_inner), lambda b, j: (j, b)),
            pl.BlockSpec((d_inner, d_model), lambda b, j: (0, 0)),
        ],
        out_specs=pl.BlockSpec((1, Lt, d_model), lambda b, j: (b, j, 0)),
        out_shape=jax.ShapeDtypeStruct((B, L, d_model), _F32),
        compiler_params=cp,
        name="mamba_out",
    )(x, W_in, y2, W_out)

    return o


# speedup vs baseline: 1.1052x; 1.0325x over previous
from functools import partial

import jax
import jax.numpy as jnp
from jax.experimental import pallas as pl
from jax.experimental.pallas import tpu as pltpu

_F32 = jnp.float32
_LOG2E = 1.4426950408889634
_LN2 = 0.6931471805599453


def _silu(v):
    return v / (1.0 + jnp.exp2(v * (-_LOG2E)))


def _softplus(v):
    return jnp.log2(1.0 + jnp.exp2(v * _LOG2E)) * _LN2


def _front_kernel(x_ref, Wu_ref, Wx_ref, Wdt_ref, cw_ref, cb_ref, bdt_ref,
                  u_ref, dt_ref, BT_ref, CT_ref, carry_ref,
                  *, dt_rank, n_state, d_conv):
    j = pl.program_id(1)

    @pl.when(j == 0)
    def _():
        carry_ref[...] = jnp.zeros_like(carry_ref)

    x = x_ref[0]
    u_raw = jnp.dot(x, Wu_ref[...], preferred_element_type=_F32)

    Lt = u_raw.shape[0]
    full = jnp.concatenate([carry_ref[...], u_raw], axis=0)
    carry_ref[...] = full[Lt:Lt + 8]
    acc = jnp.zeros_like(u_raw) + cb_ref[...]
    for k in range(d_conv):
        off = 8 - (d_conv - 1) + k
        acc = acc + full[off:off + Lt] * cw_ref[k:k + 1, :]
    u = _silu(acc)
    u_ref[...] = u

    xdbl = jnp.dot(u, Wx_ref[...], preferred_element_type=_F32)
    dt_low = xdbl[:, :dt_rank]
    dtv = _softplus(
        jnp.dot(dt_low, Wdt_ref[...], preferred_element_type=_F32) + bdt_ref[...])
    dt_ref[...] = dtv
    BT_ref[...] = xdbl[:, dt_rank:dt_rank + n_state].T
    CT_ref[...] = xdbl[:, dt_rank + n_state:dt_rank + 2 * n_state].T


def _scan_kernel(u_ref, dt_ref, BT_ref, CT_ref, A_ref, D_ref,
                 y_ref, h_ref):
    j = pl.program_id(2)

    @pl.when(j == 0)
    def _():
        h_ref[...] = jnp.zeros_like(h_ref)

    u = u_ref[...]
    dt = dt_ref[...]
    A = A_ref[...]
    dtu = dt * u
    BT = BT_ref[...]
    CT = CT_ref[...]
    h = h_ref[...]
    Tc = u.shape[0]
    for t in range(Tc):
        dA = jnp.exp2(dt[t:t + 1, :] * A)
        h = dA * h + dtu[t:t + 1, :] * BT[:, t:t + 1]
        y_ref[t:t + 1, :] = jnp.sum(h * CT[:, t:t + 1], axis=0, keepdims=True)
    h_ref[...] = h
    y_ref[...] = y_ref[...] + u * D_ref[...]


def _out_kernel(x_ref, Wr_ref, y_ref, Wo_ref, o_ref):
    res = jnp.dot(x_ref[0], Wr_ref[...], preferred_element_type=_F32)
    g = y_ref[...] * _silu(res)
    o_ref[0] = jnp.dot(g, Wo_ref[...], preferred_element_type=_F32)


def kernel(x, W_in, conv_w, conv_b, W_xproj, W_dt, b_dt, A_log, D, W_out):
    B, L, d_model = x.shape
    d_inner, d_conv = conv_w.shape
    dt_rank = W_dt.shape[0]
    n_state = A_log.shape[1]

    Lt = min(512, L)
    J1 = L // Lt
    Tc = min(256, L)
    J2 = L // Tc
    dblk = min(512, d_inner)
    Gd = d_inner // dblk

    cwT = conv_w.T
    cb = conv_b[None, :]
    bdt = b_dt[None, :]
    LOG2E = 1.4426950408889634
    AT = (-jnp.exp(A_log) * LOG2E).T
    Dc = jnp.concatenate([D] * B)[None, :]

    cp = pltpu.CompilerParams(
        dimension_semantics=("parallel", "arbitrary"),
        vmem_limit_bytes=56 * 1024 * 1024,
    )

    u_c, dt_c, BT, CT = pl.pallas_call(
        partial(_front_kernel, dt_rank=dt_rank, n_state=n_state, d_conv=d_conv),
        grid=(B, J1),
        in_specs=[
            pl.BlockSpec((1, Lt, d_model), lambda b, j: (b, j, 0)),
            pl.BlockSpec((d_model, d_inner), lambda b, j: (0, 0)),
            pl.BlockSpec((d_inner, dt_rank + 2 * n_state), lambda b, j: (0, 0)),
            pl.BlockSpec((dt_rank, d_inner), lambda b, j: (0, 0)),
            pl.BlockSpec((d_conv, d_inner), lambda b, j: (0, 0)),
            pl.BlockSpec((1, d_inner), lambda b, j: (0, 0)),
            pl.BlockSpec((1, d_inner), lambda b, j: (0, 0)),
        ],
        out_specs=[
            pl.BlockSpec((Lt, d_inner), lambda b, j: (j, b)),
            pl.BlockSpec((Lt, d_inner), lambda b, j: (j, b)),
            pl.BlockSpec((n_state, Lt), lambda b, j: (b, j)),
            pl.BlockSpec((n_state, Lt), lambda b, j: (b, j)),
        ],
        out_shape=[
            jax.ShapeDtypeStruct((L, B * d_inner), _F32),
            jax.ShapeDtypeStruct((L, B * d_inner), _F32),
            jax.ShapeDtypeStruct((B * n_state, L), _F32),
            jax.ShapeDtypeStruct((B * n_state, L), _F32),
        ],
        scratch_shapes=[pltpu.VMEM((8, d_inner), _F32)],
        compiler_params=cp,
        name="mamba_front",
    )(x, W_in, W_xproj, W_dt, cwT, cb, bdt)

    Gh = (B * Gd) // 2
    cp_scan = pltpu.CompilerParams(
        dimension_semantics=("parallel", "arbitrary", "arbitrary"),
        vmem_limit_bytes=56 * 1024 * 1024,
    )
    y2 = pl.pallas_call(
        _scan_kernel,
        grid=(2, Gh, J2),
        in_specs=[
            pl.BlockSpec((Tc, dblk), lambda c, g, j: (j, c * Gh + g)),
            pl.BlockSpec((Tc, dblk), lambda c, g, j: (j, c * Gh + g)),
            pl.BlockSpec((n_state, Tc), lambda c, g, j: ((c * Gh + g) // Gd, j)),
            pl.BlockSpec((n_state, Tc), lambda c, g, j: ((c * Gh + g) // Gd, j)),
            pl.BlockSpec((n_state, dblk), lambda c, g, j: (0, (c * Gh + g) % Gd)),
            pl.BlockSpec((1, dblk), lambda c, g, j: (0, c * Gh + g)),
        ],
        out_specs=pl.BlockSpec((Tc, dblk), lambda c, g, j: (j, c * Gh + g)),
        out_shape=jax.ShapeDtypeStruct((L, B * d_inner), _F32),
        scratch_shapes=[pltpu.VMEM((n_state, dblk), _F32)],
        compiler_params=cp_scan,
        name="mamba_scan",
    )(u_c, dt_c, BT, CT, AT, Dc)

    o = pl.pallas_call(
        _out_kernel,
        grid=(B, J1),
        in_specs=[
            pl.BlockSpec((1, Lt, d_model), lambda b, j: (b, j, 0)),
            pl.BlockSpec((d_model, d_inner), lambda b, j: (0, 1)),
            pl.BlockSpec((Lt, d_inner), lambda b, j: (j, b)),
            pl.BlockSpec((d_inner, d_model), lambda b, j: (0, 0)),
        ],
        out_specs=pl.BlockSpec((1, Lt, d_model), lambda b, j: (b, j, 0)),
        out_shape=jax.ShapeDtypeStruct((B, L, d_model), _F32),
        compiler_params=cp,
        name="mamba_out",
    )(x, W_in, y2, W_out)

    return o
